# baseline (speedup 1.0000x reference)
import functools

import jax
import jax.numpy as jnp
from jax import lax
from jax.experimental import pallas as pl
from jax.experimental.pallas import tpu as pltpu
from jax.experimental.pallas import tpu_sc as plsc

MARGIN_ = 0.2

B = 16384
D = 128
NW = 32
BPW = B // NW
C = 128
NCHUNK = BPW // C
NBUF = 2

_mesh = plsc.VectorSubcoreMesh(core_axis_name="c", subcore_axis_name="s")


def _sqrt16(x):
    xs = jnp.maximum(x, 1e-20)
    i = plsc.bitcast(xs, jnp.int32)
    i = 0x5F3759DF - lax.shift_right_logical(i, 1)
    y = plsc.bitcast(i, jnp.float32)
    for _ in range(3):
        y = y * (1.5 - 0.5 * xs * y * y)
    return x * y


@functools.partial(
    pl.kernel,
    mesh=_mesh,
    compiler_params=pltpu.CompilerParams(needs_layout_passes=False),
    out_type=jax.ShapeDtypeStruct((NW * 16,), jnp.float32),
    scratch_types=(
        [pltpu.VMEM((3 * C,), jnp.int32)] * NBUF
        + [pltpu.VMEM((3 * C, D), jnp.float32)] * NBUF
        + [pltpu.VMEM((16,), jnp.float32)]
        + [pltpu.SemaphoreType.DMA] * NBUF
    ),
)
def _sc_loss(trip_hbm, emb_hbm, out_hbm, *refs):
    ix = refs[0:NBUF]
    rows = refs[NBUF:2 * NBUF]
    acc_v = refs[2 * NBUF]
    sems = refs[2 * NBUF + 1:]
    wid = lax.axis_index("s") * 2 + lax.axis_index("c")
    base = wid * BPW
    lane = lax.iota(jnp.int32, 16)

    def stage(c):
        b = c % NBUF
        off = base + c * C
        pltpu.sync_copy(trip_hbm.at[pl.ds(off, C)], ix[b].at[pl.ds(0, C)])
        pltpu.sync_copy(trip_hbm.at[pl.ds(B + off, C)],
                        ix[b].at[pl.ds(C, C)])
        pltpu.sync_copy(trip_hbm.at[pl.ds(2 * B + off, C)],
                        ix[b].at[pl.ds(2 * C, C)])
        return pltpu.async_copy(emb_hbm.at[ix[b]], rows[b], sems[b])

    acc_v[...] = jnp.zeros((16,), jnp.float32)
    pending = [stage(c) for c in range(NBUF - 1)]
    for c in range(NCHUNK):
        if c + NBUF - 1 < NCHUNK:
            pending.append(stage(c + NBUF - 1))
        pending.pop(0).wait()
        b = c % NBUF
        rows_b = rows[b]

        def body(g, _, rows_b=rows_b):
            def tbody(k, vs, rows_b=rows_b, g=g):
                vp2, vn2 = vs
                t = g * 16 + k

                def jbody(j, accs, rows_b=rows_b, t=t):
                    acc_p, acc_n = accs
                    av = rows_b[t, pl.ds(j * 16, 16)]
                    pv = rows_b[C + t, pl.ds(j * 16, 16)]
                    nv = rows_b[2 * C + t, pl.ds(j * 16, 16)]
                    dp = av - pv
                    dn = av - nv
                    return (acc_p + dp * dp, acc_n + dn * dn)

                z16 = jnp.zeros((16,), jnp.float32)
                acc_p, acc_n = lax.fori_loop(0, D // 16, jbody, (z16, z16))
                vp2 = jnp.where(lane == k, jnp.sum(acc_p), vp2)
                vn2 = jnp.where(lane == k, jnp.sum(acc_n), vn2)
                return (vp2, vn2)

            zero16 = jnp.zeros((16,), jnp.float32)
            vp2, vn2 = lax.fori_loop(0, 16, tbody, (zero16, zero16))
            loss = jnp.maximum(_sqrt16(vp2) - _sqrt16(vn2) + MARGIN_, 0.0)
            acc_v[...] = acc_v[...] + loss
            return 0

        lax.fori_loop(0, C // 16, body, 0)

    pltpu.sync_copy(acc_v, out_hbm.at[pl.ds(wid * 16, 16)])


def _tc_finish_body(part_ref, out_ref):
    out_ref[0, 0] = jnp.sum(part_ref[...]) * (1.0 / B)


_tc_finish = pl.pallas_call(
    _tc_finish_body,
    out_shape=jax.ShapeDtypeStruct((1, 1), jnp.float32),
    out_specs=pl.BlockSpec(memory_space=pltpu.SMEM),
)


def kernel(triplets, embeddings):
    triplets = triplets.astype(jnp.int32)
    part = _sc_loss(triplets.T.reshape(-1), embeddings)
    return _tc_finish(part).reshape(())


# speedup vs baseline: 1.0195x; 1.0195x over previous
import functools

import jax
import jax.numpy as jnp
from jax import lax
from jax.experimental import pallas as pl
from jax.experimental.pallas import tpu as pltpu
from jax.experimental.pallas import tpu_sc as plsc

MARGIN_ = 0.2

B = 16384
D = 128
NW = 32
BPW = B // NW
C = 128
NCHUNK = BPW // C
NBUF = 2

_mesh = plsc.VectorSubcoreMesh(core_axis_name="c", subcore_axis_name="s")


def _sqrt16(x):
    xs = jnp.maximum(x, 1e-20)
    i = plsc.bitcast(xs, jnp.int32)
    i = 0x5F3759DF - lax.shift_right_logical(i, 1)
    y = plsc.bitcast(i, jnp.float32)
    for _ in range(3):
        y = y * (1.5 - 0.5 * xs * y * y)
    return x * y


@functools.partial(
    pl.kernel,
    mesh=_mesh,
    compiler_params=pltpu.CompilerParams(needs_layout_passes=False),
    out_type=jax.ShapeDtypeStruct((NW * 16,), jnp.float32),
    scratch_types=(
        [pltpu.VMEM((3 * C,), jnp.int32)] * NBUF
        + [pltpu.VMEM((3 * C, D), jnp.float32)] * NBUF
        + [pltpu.VMEM((16,), jnp.float32)]
        + [pltpu.SemaphoreType.DMA] * NBUF
    ),
)
def _sc_loss(trip_hbm, emb_hbm, out_hbm, *refs):
    ix = refs[0:NBUF]
    rows = refs[NBUF:2 * NBUF]
    acc_v = refs[2 * NBUF]
    sems = refs[2 * NBUF + 1:]
    wid = lax.axis_index("s") * 2 + lax.axis_index("c")
    base = wid * BPW
    lane = lax.iota(jnp.int32, 16)

    def stage(c, b):
        off = base + c * C
        pltpu.sync_copy(trip_hbm.at[pl.ds(off, C)], ix[b].at[pl.ds(0, C)])
        pltpu.sync_copy(trip_hbm.at[pl.ds(B + off, C)],
                        ix[b].at[pl.ds(C, C)])
        pltpu.sync_copy(trip_hbm.at[pl.ds(2 * B + off, C)],
                        ix[b].at[pl.ds(2 * C, C)])
        return pltpu.async_copy(emb_hbm.at[ix[b]], rows[b], sems[b])

    acc_v[...] = jnp.zeros((16,), jnp.float32)
    stage(0, 0)

    def chunk_pair(cc, _):
        for b in range(NBUF):
            c = cc * NBUF + b
            nb = (b + 1) % NBUF

            @pl.when(c + 1 < NCHUNK)
            def _(c=c, nb=nb):
                stage(c + 1, nb)

            pltpu.make_async_copy(emb_hbm.at[ix[b]], rows[b], sems[b]).wait()
            compute(c, b)
        return 0

    def compute(c, b):
        rows_b = rows[b]

        def body(g, _, rows_b=rows_b):
            def tbody(k, vs, rows_b=rows_b, g=g):
                vp2, vn2 = vs
                t = g * 16 + k

                def jbody(j, accs, rows_b=rows_b, t=t):
                    acc_p, acc_n = accs
                    av = rows_b[t, pl.ds(j * 16, 16)]
                    pv = rows_b[C + t, pl.ds(j * 16, 16)]
                    nv = rows_b[2 * C + t, pl.ds(j * 16, 16)]
                    dp = av - pv
                    dn = av - nv
                    return (acc_p + dp * dp, acc_n + dn * dn)

                z16 = jnp.zeros((16,), jnp.float32)
                acc_p, acc_n = lax.fori_loop(0, D // 16, jbody, (z16, z16))
                vp2 = jnp.where(lane == k, jnp.sum(acc_p), vp2)
                vn2 = jnp.where(lane == k, jnp.sum(acc_n), vn2)
                return (vp2, vn2)

            zero16 = jnp.zeros((16,), jnp.float32)
            vp2, vn2 = lax.fori_loop(0, 16, tbody, (zero16, zero16))
            loss = jnp.maximum(_sqrt16(vp2) - _sqrt16(vn2) + MARGIN_, 0.0)
            acc_v[...] = acc_v[...] + loss
            return 0

        lax.fori_loop(0, C // 16, body, 0)

    lax.fori_loop(0, NCHUNK // NBUF, chunk_pair, 0)
    pltpu.sync_copy(acc_v, out_hbm.at[pl.ds(wid * 16, 16)])


def _tc_finish_body(part_ref, out_ref):
    out_ref[0, 0] = jnp.sum(part_ref[...]) * (1.0 / B)


_tc_finish = pl.pallas_call(
    _tc_finish_body,
    out_shape=jax.ShapeDtypeStruct((1, 1), jnp.float32),
    out_specs=pl.BlockSpec(memory_space=pltpu.SMEM),
)


def kernel(triplets, embeddings):
    triplets = triplets.astype(jnp.int32)
    part = _sc_loss(triplets.T.reshape(-1), embeddings)
    return _tc_finish(part).reshape(())
